# baseline (speedup 1.0000x reference)
import functools

import jax
import jax.numpy as jnp
from jax import lax
from jax.experimental import pallas as pl
from jax.experimental.pallas import tpu as pltpu
from jax.experimental.pallas import tpu_sc as plsc

N_NODES = 10000
N_EDGES = 320000
NPAD = 10240
D_HIDDEN = 128
D_OUT_PAD = 64

NC = 2
NS = 16
NW = NC * NS

K_CHUNK = 128
EDGES_PER_WORKER = 10112
E_PAD = NW * EDGES_PER_WORKER
N_CHUNKS = EDGES_PER_WORKER // K_CHUNK
TOT_CHUNKS = E_PAD // K_CHUNK
CH_C0 = 98
CH_C1 = 60

DEG_PER_WORKER = N_EDGES // NW


def _deg_body(dst_hbm, deg_out, dst_v, deg_v):
    cid = lax.axis_index("c")
    sid = lax.axis_index("s")
    wid = sid * NC + cid

    pltpu.sync_copy(dst_hbm.at[pl.ds(wid * DEG_PER_WORKER, DEG_PER_WORKER)],
                    dst_v)

    def zero(i, _):
        deg_v[pl.ds(i * 16, 16)] = jnp.zeros((16,), jnp.float32)
        return _
    lax.fori_loop(0, NPAD // 16, zero, None)

    ones = jnp.ones((16,), jnp.float32)

    def body(i, _):
        idx = dst_v[pl.ds(i * 16, 16)]
        plsc.addupdate_scatter(deg_v, [idx], ones)
        return _
    lax.fori_loop(0, DEG_PER_WORKER // 16, body, None)

    pltpu.sync_copy(deg_v, deg_out.at[wid])


def _sc_deg(dst):
    return pl.kernel(
        _deg_body,
        out_type=jax.ShapeDtypeStruct((NW, NPAD), jnp.float32),
        mesh=plsc.VectorSubcoreMesh(core_axis_name="c", subcore_axis_name="s"),
        scratch_types=[
            pltpu.VMEM((DEG_PER_WORKER,), jnp.int32),
            pltpu.VMEM((NPAD,), jnp.float32),
        ],
        compiler_params=pltpu.CompilerParams(needs_layout_passes=False),
    )(dst)


def _mp_body(d, y_hbm, src2_hbm, dst2_hbm, zeros_hbm, out_hbm,
             src_v, dst_v, rows_v, acc_sh, sem):
    cid = lax.axis_index("c")
    sid = lax.axis_index("s")
    wid = sid * NC + cid
    rpt = NPAD // NS

    pltpu.sync_copy(zeros_hbm.at[pl.ds(sid * rpt, rpt)],
                    acc_sh.at[pl.ds(sid * rpt, rpt)])
    start = jnp.where(cid == 0, sid * CH_C0, NS * CH_C0 + sid * CH_C1)
    count = jnp.where(cid == 0, CH_C0, CH_C1)
    pltpu.sync_copy(dst2_hbm.at[pl.ds(start, CH_C0)], dst_v)
    pltpu.sync_copy(src2_hbm.at[pl.ds(start, CH_C0)], src_v)
    plsc.subcore_barrier()

    def body(i, _):
        pltpu.async_copy(y_hbm.at[src_v.at[i]], rows_v, sem).wait()
        pltpu.sync_copy(rows_v, acc_sh.at[dst_v.at[i]], add=True)
        return _
    lax.fori_loop(0, count, body, None)

    plsc.subcore_barrier()
    pltpu.sync_copy(
        acc_sh.at[pl.ds(sid * rpt, rpt)],
        out_hbm.at[pl.ds(cid * NPAD + sid * rpt, rpt)])


def _sc_mp(y, src, dst2, zeros_nd, d):
    return pl.kernel(
        functools.partial(_mp_body, d),
        out_type=jax.ShapeDtypeStruct((NC * NPAD, d), jnp.float32),
        mesh=plsc.VectorSubcoreMesh(core_axis_name="c", subcore_axis_name="s"),
        scratch_types=[
            pltpu.VMEM((CH_C0, K_CHUNK), jnp.int32),
            pltpu.VMEM((CH_C0, K_CHUNK), jnp.int32),
            pltpu.VMEM((K_CHUNK, d), jnp.float32),
            pltpu.VMEM_SHARED((NPAD, d), jnp.float32),
            pltpu.SemaphoreType.DMA,
        ],
        compiler_params=pltpu.CompilerParams(use_tc_tiling_on_sc=False),
    )(y, src, dst2, zeros_nd)


def _mm1_body(x_ref, w_ref, h_ref):
    h_ref[...] = jnp.dot(x_ref[...], w_ref[...],
                         preferred_element_type=jnp.float32)


def _scale1_body(deg_ref, h_ref, y_ref, dinv_ref):
    deg = jnp.sum(deg_ref[...], axis=0) + 1.0
    dinv = lax.rsqrt(jnp.maximum(deg, 1.0))
    dinv_ref[...] = dinv
    y_ref[...] = h_ref[...] * dinv[:N_NODES, None]


def _mid_body(acc_ref, y1_ref, dinv_ref, w2_ref, y2_ref):
    dinv = dinv_ref[...][:N_NODES, None]
    s = acc_ref[:N_NODES, :] + acc_ref[NPAD:NPAD + N_NODES, :] + y1_ref[...]
    h1 = jnp.maximum(s * dinv, 0.0)
    y2_ref[...] = jnp.dot(h1, w2_ref[...],
                          preferred_element_type=jnp.float32) * dinv


def _final_body(acc_ref, y2_ref, dinv_ref, out_ref):
    dinv = dinv_ref[...][:N_NODES, None]
    s = acc_ref[:N_NODES, :] + acc_ref[NPAD:NPAD + N_NODES, :] + y2_ref[...]
    out_ref[...] = s * dinv


def _tc_call(body, out_shapes, *args):
    return pl.pallas_call(
        body,
        out_shape=out_shapes,
    )(*args)


def kernel(x, edge_index, W1, W2):
    src = edge_index[0]
    dst = edge_index[1]
    pad = E_PAD - N_EDGES
    src_p = jnp.concatenate([src, jnp.zeros((pad,), jnp.int32)])
    dst_p = jnp.concatenate([dst, jnp.full((pad,), N_NODES, jnp.int32)])
    extra = jnp.zeros((NS * (CH_C0 + CH_C1) + CH_C0 - TOT_CHUNKS, K_CHUNK),
                      jnp.int32)
    src2 = jnp.concatenate([src_p.reshape(-1, K_CHUNK), extra])
    dst2 = jnp.concatenate([dst_p.reshape(-1, K_CHUNK), extra])

    zeros128 = jnp.zeros((NPAD, D_HIDDEN), jnp.float32)
    zeros64 = jnp.zeros((NPAD, D_OUT_PAD), jnp.float32)
    W2p = jnp.pad(W2, ((0, 0), (0, D_OUT_PAD - W2.shape[1])))

    deg_all = _sc_deg(dst)
    h = _tc_call(_mm1_body, jax.ShapeDtypeStruct((N_NODES, D_HIDDEN),
                                                 jnp.float32), x, W1)
    y1, dinv = _tc_call(
        _scale1_body,
        [jax.ShapeDtypeStruct((N_NODES, D_HIDDEN), jnp.float32),
         jax.ShapeDtypeStruct((NPAD,), jnp.float32)],
        deg_all, h)

    acc1 = _sc_mp(y1, src2, dst2, zeros128, D_HIDDEN)
    y2 = _tc_call(_mid_body,
                  jax.ShapeDtypeStruct((N_NODES, D_OUT_PAD), jnp.float32),
                  acc1, y1, dinv, W2p)

    acc2 = _sc_mp(y2, src2, dst2, zeros64, D_OUT_PAD)
    out = _tc_call(_final_body,
                   jax.ShapeDtypeStruct((N_NODES, D_OUT_PAD), jnp.float32),
                   acc2, y2, dinv)
    return out[:, :W2.shape[1]]


# speedup vs baseline: 1.0998x; 1.0998x over previous
import functools

import jax
import jax.numpy as jnp
from jax import lax
from jax.experimental import pallas as pl
from jax.experimental.pallas import tpu as pltpu
from jax.experimental.pallas import tpu_sc as plsc

N_NODES = 10000
N_EDGES = 320000
NPAD = 10240
D_HIDDEN = 128
D_OUT_PAD = 64

NC = 2
NS = 16
NW = NC * NS

K_CHUNK = 128
EDGES_PER_WORKER = 10112
E_PAD = NW * EDGES_PER_WORKER
N_CHUNKS = EDGES_PER_WORKER // K_CHUNK
TOT_CHUNKS = E_PAD // K_CHUNK
CH_C0 = 88
CH_C1 = 70

DEG_PER_WORKER = N_EDGES // NW


def _deg_body(dst_hbm, deg_out, dst_v, deg_v):
    cid = lax.axis_index("c")
    sid = lax.axis_index("s")
    wid = sid * NC + cid

    pltpu.sync_copy(dst_hbm.at[pl.ds(wid * DEG_PER_WORKER, DEG_PER_WORKER)],
                    dst_v)

    def zero(i, _):
        deg_v[pl.ds(i * 16, 16)] = jnp.zeros((16,), jnp.float32)
        return _
    lax.fori_loop(0, NPAD // 16, zero, None)

    ones = jnp.ones((16,), jnp.float32)

    def body(i, _):
        idx = dst_v[pl.ds(i * 16, 16)]
        plsc.addupdate_scatter(deg_v, [idx], ones)
        return _
    lax.fori_loop(0, DEG_PER_WORKER // 16, body, None)

    pltpu.sync_copy(deg_v, deg_out.at[wid])


def _sc_deg(dst):
    return pl.kernel(
        _deg_body,
        out_type=jax.ShapeDtypeStruct((NW, NPAD), jnp.float32),
        mesh=plsc.VectorSubcoreMesh(core_axis_name="c", subcore_axis_name="s"),
        scratch_types=[
            pltpu.VMEM((DEG_PER_WORKER,), jnp.int32),
            pltpu.VMEM((NPAD,), jnp.float32),
        ],
        compiler_params=pltpu.CompilerParams(needs_layout_passes=False),
    )(dst)


def _mp_body(d, y_hbm, src2_hbm, dst2_hbm, zeros_hbm, out_hbm,
             src_v, dst_v, rows_v, acc_sh, sem):
    cid = lax.axis_index("c")
    sid = lax.axis_index("s")
    wid = sid * NC + cid
    rpt = NPAD // NS

    pltpu.sync_copy(zeros_hbm.at[pl.ds(sid * rpt, rpt)],
                    acc_sh.at[pl.ds(sid * rpt, rpt)])
    start = jnp.where(cid == 0, sid * CH_C0, NS * CH_C0 + sid * CH_C1)
    count = jnp.where(cid == 0, CH_C0, CH_C1)
    pltpu.sync_copy(dst2_hbm.at[pl.ds(start, CH_C0)], dst_v)
    pltpu.sync_copy(src2_hbm.at[pl.ds(start, CH_C0)], src_v)
    plsc.subcore_barrier()

    def body(i, _):
        pltpu.async_copy(y_hbm.at[src_v.at[i]], rows_v, sem).wait()
        pltpu.sync_copy(rows_v, acc_sh.at[dst_v.at[i]], add=True)
        return _
    lax.fori_loop(0, count, body, None)

    plsc.subcore_barrier()
    pltpu.sync_copy(
        acc_sh.at[pl.ds(sid * rpt, rpt)],
        out_hbm.at[pl.ds(cid * NPAD + sid * rpt, rpt)])


def _sc_mp(y, src, dst2, zeros_nd, d):
    return pl.kernel(
        functools.partial(_mp_body, d),
        out_type=jax.ShapeDtypeStruct((NC * NPAD, d), jnp.float32),
        mesh=plsc.VectorSubcoreMesh(core_axis_name="c", subcore_axis_name="s"),
        scratch_types=[
            pltpu.VMEM((CH_C0, K_CHUNK), jnp.int32),
            pltpu.VMEM((CH_C0, K_CHUNK), jnp.int32),
            pltpu.VMEM((K_CHUNK, d), jnp.float32),
            pltpu.VMEM_SHARED((NPAD, d), jnp.float32),
            pltpu.SemaphoreType.DMA,
        ],
        compiler_params=pltpu.CompilerParams(use_tc_tiling_on_sc=False),
    )(y, src, dst2, zeros_nd)


def _mm1_body(x_ref, w_ref, h_ref):
    h_ref[...] = jnp.dot(x_ref[...], w_ref[...],
                         preferred_element_type=jnp.float32)


def _scale1_body(deg_ref, h_ref, y_ref, dinv_ref):
    deg = jnp.sum(deg_ref[...], axis=0) + 1.0
    dinv = lax.rsqrt(jnp.maximum(deg, 1.0))
    dinv_ref[...] = dinv
    y_ref[...] = h_ref[...] * dinv[:N_NODES, None]


def _mid_body(acc_ref, y1_ref, dinv_ref, w2_ref, y2_ref):
    dinv = dinv_ref[...][:N_NODES, None]
    s = acc_ref[:N_NODES, :] + acc_ref[NPAD:NPAD + N_NODES, :] + y1_ref[...]
    h1 = jnp.maximum(s * dinv, 0.0)
    y2_ref[...] = jnp.dot(h1, w2_ref[...],
                          preferred_element_type=jnp.float32) * dinv


def _final_body(acc_ref, y2_ref, dinv_ref, out_ref):
    dinv = dinv_ref[...][:N_NODES, None]
    s = acc_ref[:N_NODES, :] + acc_ref[NPAD:NPAD + N_NODES, :] + y2_ref[...]
    out_ref[...] = s * dinv


def _tc_call(body, out_shapes, *args):
    return pl.pallas_call(
        body,
        out_shape=out_shapes,
    )(*args)


def kernel(x, edge_index, W1, W2):
    src = edge_index[0]
    dst = edge_index[1]
    pad = E_PAD - N_EDGES
    src_p = jnp.concatenate([src, jnp.zeros((pad,), jnp.int32)])
    dst_p = jnp.concatenate([dst, jnp.full((pad,), N_NODES, jnp.int32)])
    extra = jnp.zeros((NS * (CH_C0 + CH_C1) + CH_C0 - TOT_CHUNKS, K_CHUNK),
                      jnp.int32)
    src2 = jnp.concatenate([src_p.reshape(-1, K_CHUNK), extra])
    dst2 = jnp.concatenate([dst_p.reshape(-1, K_CHUNK), extra])

    zeros128 = jnp.zeros((NPAD, D_HIDDEN), jnp.float32)
    zeros64 = jnp.zeros((NPAD, D_OUT_PAD), jnp.float32)
    W2p = jnp.pad(W2, ((0, 0), (0, D_OUT_PAD - W2.shape[1])))

    deg_all = _sc_deg(dst)
    h = _tc_call(_mm1_body, jax.ShapeDtypeStruct((N_NODES, D_HIDDEN),
                                                 jnp.float32), x, W1)
    y1, dinv = _tc_call(
        _scale1_body,
        [jax.ShapeDtypeStruct((N_NODES, D_HIDDEN), jnp.float32),
         jax.ShapeDtypeStruct((NPAD,), jnp.float32)],
        deg_all, h)

    acc1 = _sc_mp(y1, src2, dst2, zeros128, D_HIDDEN)
    y2 = _tc_call(_mid_body,
                  jax.ShapeDtypeStruct((N_NODES, D_OUT_PAD), jnp.float32),
                  acc1, y1, dinv, W2p)

    acc2 = _sc_mp(y2, src2, dst2, zeros64, D_OUT_PAD)
    out = _tc_call(_final_body,
                   jax.ShapeDtypeStruct((N_NODES, D_OUT_PAD), jnp.float32),
                   acc2, y2, dinv)
    return out[:, :W2.shape[1]]


# speedup vs baseline: 1.1182x; 1.0168x over previous
import functools

import jax
import jax.numpy as jnp
from jax import lax
from jax.experimental import pallas as pl
from jax.experimental.pallas import tpu as pltpu
from jax.experimental.pallas import tpu_sc as plsc

N_NODES = 10000
N_EDGES = 320000
NPAD = 10240
D_HIDDEN = 128
D_OUT_PAD = 64

NC = 2
NS = 16
NW = NC * NS

K_CHUNK = 128
EDGES_PER_WORKER = 10112
E_PAD = NW * EDGES_PER_WORKER
N_CHUNKS = EDGES_PER_WORKER // K_CHUNK
TOT_CHUNKS = E_PAD // K_CHUNK
CH_C0 = 92
CH_C1 = 66

DEG_PER_WORKER = N_EDGES // NW


def _deg_body(dst_hbm, deg_out, dst_v, deg_v):
    cid = lax.axis_index("c")
    sid = lax.axis_index("s")
    wid = sid * NC + cid

    pltpu.sync_copy(dst_hbm.at[pl.ds(wid * DEG_PER_WORKER, DEG_PER_WORKER)],
                    dst_v)

    def zero(i, _):
        deg_v[pl.ds(i * 16, 16)] = jnp.zeros((16,), jnp.float32)
        return _
    lax.fori_loop(0, NPAD // 16, zero, None)

    ones = jnp.ones((16,), jnp.float32)

    def body(i, _):
        idx = dst_v[pl.ds(i * 16, 16)]
        plsc.addupdate_scatter(deg_v, [idx], ones)
        return _
    lax.fori_loop(0, DEG_PER_WORKER // 16, body, None)

    pltpu.sync_copy(deg_v, deg_out.at[wid])


def _sc_deg(dst):
    return pl.kernel(
        _deg_body,
        out_type=jax.ShapeDtypeStruct((NW, NPAD), jnp.float32),
        mesh=plsc.VectorSubcoreMesh(core_axis_name="c", subcore_axis_name="s"),
        scratch_types=[
            pltpu.VMEM((DEG_PER_WORKER,), jnp.int32),
            pltpu.VMEM((NPAD,), jnp.float32),
        ],
        compiler_params=pltpu.CompilerParams(needs_layout_passes=False),
    )(dst)


def _mp_body(d, y_hbm, src2_hbm, dst2_hbm, zeros_hbm, out_hbm,
             src_v, dst_v, rows_v, acc_sh, sem):
    cid = lax.axis_index("c")
    sid = lax.axis_index("s")
    wid = sid * NC + cid
    rpt = NPAD // NS

    pltpu.sync_copy(zeros_hbm.at[pl.ds(sid * rpt, rpt)],
                    acc_sh.at[pl.ds(sid * rpt, rpt)])
    start = jnp.where(cid == 0, sid * CH_C0, NS * CH_C0 + sid * CH_C1)
    count = jnp.where(cid == 0, CH_C0, CH_C1)
    pltpu.sync_copy(dst2_hbm.at[pl.ds(start, CH_C0)], dst_v)
    pltpu.sync_copy(src2_hbm.at[pl.ds(start, CH_C0)], src_v)
    plsc.subcore_barrier()

    def body(i, _):
        pltpu.async_copy(y_hbm.at[src_v.at[i]], rows_v, sem).wait()
        pltpu.sync_copy(rows_v, acc_sh.at[dst_v.at[i]], add=True)
        return _
    lax.fori_loop(0, count, body, None)

    plsc.subcore_barrier()
    pltpu.sync_copy(
        acc_sh.at[pl.ds(sid * rpt, rpt)],
        out_hbm.at[pl.ds(cid * NPAD + sid * rpt, rpt)])


def _sc_mp(y, src, dst2, zeros_nd, d):
    return pl.kernel(
        functools.partial(_mp_body, d),
        out_type=jax.ShapeDtypeStruct((NC * NPAD, d), jnp.float32),
        mesh=plsc.VectorSubcoreMesh(core_axis_name="c", subcore_axis_name="s"),
        scratch_types=[
            pltpu.VMEM((CH_C0, K_CHUNK), jnp.int32),
            pltpu.VMEM((CH_C0, K_CHUNK), jnp.int32),
            pltpu.VMEM((K_CHUNK, d), jnp.float32),
            pltpu.VMEM_SHARED((NPAD, d), jnp.float32),
            pltpu.SemaphoreType.DMA,
        ],
        compiler_params=pltpu.CompilerParams(use_tc_tiling_on_sc=False),
    )(y, src, dst2, zeros_nd)


def _mm1_body(x_ref, w_ref, h_ref):
    h_ref[...] = jnp.dot(x_ref[...], w_ref[...],
                         preferred_element_type=jnp.float32)


def _scale1_body(deg_ref, h_ref, y_ref, dinv_ref):
    deg = jnp.sum(deg_ref[...], axis=0) + 1.0
    dinv = lax.rsqrt(jnp.maximum(deg, 1.0))
    dinv_ref[...] = dinv
    y_ref[...] = h_ref[...] * dinv[:N_NODES, None]


def _mid_body(acc_ref, y1_ref, dinv_ref, w2_ref, y2_ref):
    dinv = dinv_ref[...][:N_NODES, None]
    s = acc_ref[:N_NODES, :] + acc_ref[NPAD:NPAD + N_NODES, :] + y1_ref[...]
    h1 = jnp.maximum(s * dinv, 0.0)
    y2_ref[...] = jnp.dot(h1, w2_ref[...],
                          preferred_element_type=jnp.float32) * dinv


def _final_body(acc_ref, y2_ref, dinv_ref, out_ref):
    dinv = dinv_ref[...][:N_NODES, None]
    s = acc_ref[:N_NODES, :] + acc_ref[NPAD:NPAD + N_NODES, :] + y2_ref[...]
    out_ref[...] = s * dinv


def _tc_call(body, out_shapes, *args):
    return pl.pallas_call(
        body,
        out_shape=out_shapes,
    )(*args)


def kernel(x, edge_index, W1, W2):
    src = edge_index[0]
    dst = edge_index[1]
    pad = E_PAD - N_EDGES
    src_p = jnp.concatenate([src, jnp.zeros((pad,), jnp.int32)])
    dst_p = jnp.concatenate([dst, jnp.full((pad,), N_NODES, jnp.int32)])
    extra = jnp.zeros((NS * (CH_C0 + CH_C1) + CH_C0 - TOT_CHUNKS, K_CHUNK),
                      jnp.int32)
    src2 = jnp.concatenate([src_p.reshape(-1, K_CHUNK), extra])
    dst2 = jnp.concatenate([dst_p.reshape(-1, K_CHUNK), extra])

    zeros128 = jnp.zeros((NPAD, D_HIDDEN), jnp.float32)
    zeros64 = jnp.zeros((NPAD, D_OUT_PAD), jnp.float32)
    W2p = jnp.pad(W2, ((0, 0), (0, D_OUT_PAD - W2.shape[1])))

    deg_all = _sc_deg(dst)
    h = _tc_call(_mm1_body, jax.ShapeDtypeStruct((N_NODES, D_HIDDEN),
                                                 jnp.float32), x, W1)
    y1, dinv = _tc_call(
        _scale1_body,
        [jax.ShapeDtypeStruct((N_NODES, D_HIDDEN), jnp.float32),
         jax.ShapeDtypeStruct((NPAD,), jnp.float32)],
        deg_all, h)

    acc1 = _sc_mp(y1, src2, dst2, zeros128, D_HIDDEN)
    y2 = _tc_call(_mid_body,
                  jax.ShapeDtypeStruct((N_NODES, D_OUT_PAD), jnp.float32),
                  acc1, y1, dinv, W2p)

    acc2 = _sc_mp(y2, src2, dst2, zeros64, D_OUT_PAD)
    out = _tc_call(_final_body,
                   jax.ShapeDtypeStruct((N_NODES, D_OUT_PAD), jnp.float32),
                   acc2, y2, dinv)
    return out[:, :W2.shape[1]]


# speedup vs baseline: 1.1490x; 1.0276x over previous
import functools

import jax
import jax.numpy as jnp
from jax import lax
from jax.experimental import pallas as pl
from jax.experimental.pallas import tpu as pltpu
from jax.experimental.pallas import tpu_sc as plsc

N_NODES = 10000
N_EDGES = 320000
NPAD = 10240
D_HIDDEN = 128
D_OUT_PAD = 48

NC = 2
NS = 16
NW = NC * NS

K_CHUNK = 128
EDGES_PER_WORKER = 10112
E_PAD = NW * EDGES_PER_WORKER
N_CHUNKS = EDGES_PER_WORKER // K_CHUNK
TOT_CHUNKS = E_PAD // K_CHUNK
CH_C0 = 92
CH_C1 = 66

DEG_PER_WORKER = N_EDGES // NW


def _deg_body(dst_hbm, deg_out, dst_v, deg_v):
    cid = lax.axis_index("c")
    sid = lax.axis_index("s")
    wid = sid * NC + cid

    pltpu.sync_copy(dst_hbm.at[pl.ds(wid * DEG_PER_WORKER, DEG_PER_WORKER)],
                    dst_v)

    def zero(i, _):
        deg_v[pl.ds(i * 16, 16)] = jnp.zeros((16,), jnp.float32)
        return _
    lax.fori_loop(0, NPAD // 16, zero, None)

    ones = jnp.ones((16,), jnp.float32)

    def body(i, _):
        idx = dst_v[pl.ds(i * 16, 16)]
        plsc.addupdate_scatter(deg_v, [idx], ones)
        return _
    lax.fori_loop(0, DEG_PER_WORKER // 16, body, None)

    pltpu.sync_copy(deg_v, deg_out.at[wid])


def _sc_deg(dst):
    return pl.kernel(
        _deg_body,
        out_type=jax.ShapeDtypeStruct((NW, NPAD), jnp.float32),
        mesh=plsc.VectorSubcoreMesh(core_axis_name="c", subcore_axis_name="s"),
        scratch_types=[
            pltpu.VMEM((DEG_PER_WORKER,), jnp.int32),
            pltpu.VMEM((NPAD,), jnp.float32),
        ],
        compiler_params=pltpu.CompilerParams(needs_layout_passes=False),
    )(dst)


def _mp_body(d, y_hbm, src2_hbm, dst2_hbm, zeros_hbm, out_hbm,
             src_v, dst_v, rows_v, acc_sh, sem):
    cid = lax.axis_index("c")
    sid = lax.axis_index("s")
    wid = sid * NC + cid
    rpt = NPAD // NS

    pltpu.sync_copy(zeros_hbm.at[pl.ds(sid * rpt, rpt)],
                    acc_sh.at[pl.ds(sid * rpt, rpt)])
    start = jnp.where(cid == 0, sid * CH_C0, NS * CH_C0 + sid * CH_C1)
    count = jnp.where(cid == 0, CH_C0, CH_C1)
    pltpu.sync_copy(dst2_hbm.at[pl.ds(start, CH_C0)], dst_v)
    pltpu.sync_copy(src2_hbm.at[pl.ds(start, CH_C0)], src_v)
    plsc.subcore_barrier()

    def body(i, _):
        pltpu.async_copy(y_hbm.at[src_v.at[i]], rows_v, sem).wait()
        pltpu.sync_copy(rows_v, acc_sh.at[dst_v.at[i]], add=True)
        return _
    lax.fori_loop(0, count, body, None)

    plsc.subcore_barrier()
    pltpu.sync_copy(
        acc_sh.at[pl.ds(sid * rpt, rpt)],
        out_hbm.at[pl.ds(cid * NPAD + sid * rpt, rpt)])


def _sc_mp(y, src, dst2, zeros_nd, d):
    return pl.kernel(
        functools.partial(_mp_body, d),
        out_type=jax.ShapeDtypeStruct((NC * NPAD, d), jnp.float32),
        mesh=plsc.VectorSubcoreMesh(core_axis_name="c", subcore_axis_name="s"),
        scratch_types=[
            pltpu.VMEM((CH_C0, K_CHUNK), jnp.int32),
            pltpu.VMEM((CH_C0, K_CHUNK), jnp.int32),
            pltpu.VMEM((K_CHUNK, d), jnp.float32),
            pltpu.VMEM_SHARED((NPAD, d), jnp.float32),
            pltpu.SemaphoreType.DMA,
        ],
        compiler_params=pltpu.CompilerParams(use_tc_tiling_on_sc=False),
    )(y, src, dst2, zeros_nd)


def _mm1_body(x_ref, w_ref, h_ref):
    h_ref[...] = jnp.dot(x_ref[...], w_ref[...],
                         preferred_element_type=jnp.float32)


def _scale1_body(deg_ref, h_ref, y_ref, dinv_ref):
    deg = jnp.sum(deg_ref[...], axis=0) + 1.0
    dinv = lax.rsqrt(jnp.maximum(deg, 1.0))
    dinv_ref[...] = dinv
    y_ref[...] = h_ref[...] * dinv[:N_NODES, None]


def _mid_body(acc_ref, y1_ref, dinv_ref, w2_ref, y2_ref):
    dinv = dinv_ref[...][:N_NODES, None]
    s = acc_ref[:N_NODES, :] + acc_ref[NPAD:NPAD + N_NODES, :] + y1_ref[...]
    h1 = jnp.maximum(s * dinv, 0.0)
    y2_ref[...] = jnp.dot(h1, w2_ref[...],
                          preferred_element_type=jnp.float32) * dinv


def _final_body(acc_ref, y2_ref, dinv_ref, out_ref):
    dinv = dinv_ref[...][:N_NODES, None]
    s = acc_ref[:N_NODES, :] + acc_ref[NPAD:NPAD + N_NODES, :] + y2_ref[...]
    out_ref[...] = s * dinv


def _tc_call(body, out_shapes, *args):
    return pl.pallas_call(
        body,
        out_shape=out_shapes,
    )(*args)


def kernel(x, edge_index, W1, W2):
    src = edge_index[0]
    dst = edge_index[1]
    pad = E_PAD - N_EDGES
    src_p = jnp.concatenate([src, jnp.zeros((pad,), jnp.int32)])
    dst_p = jnp.concatenate([dst, jnp.full((pad,), N_NODES, jnp.int32)])
    extra = jnp.zeros((NS * (CH_C0 + CH_C1) + CH_C0 - TOT_CHUNKS, K_CHUNK),
                      jnp.int32)
    src2 = jnp.concatenate([src_p.reshape(-1, K_CHUNK), extra])
    dst2 = jnp.concatenate([dst_p.reshape(-1, K_CHUNK), extra])

    zeros128 = jnp.zeros((NPAD, D_HIDDEN), jnp.float32)
    zeros64 = jnp.zeros((NPAD, D_OUT_PAD), jnp.float32)
    W2p = jnp.pad(W2, ((0, 0), (0, D_OUT_PAD - W2.shape[1])))

    deg_all = _sc_deg(dst)
    h = _tc_call(_mm1_body, jax.ShapeDtypeStruct((N_NODES, D_HIDDEN),
                                                 jnp.float32), x, W1)
    y1, dinv = _tc_call(
        _scale1_body,
        [jax.ShapeDtypeStruct((N_NODES, D_HIDDEN), jnp.float32),
         jax.ShapeDtypeStruct((NPAD,), jnp.float32)],
        deg_all, h)

    acc1 = _sc_mp(y1, src2, dst2, zeros128, D_HIDDEN)
    y2 = _tc_call(_mid_body,
                  jax.ShapeDtypeStruct((N_NODES, D_OUT_PAD), jnp.float32),
                  acc1, y1, dinv, W2p)

    acc2 = _sc_mp(y2, src2, dst2, zeros64, D_OUT_PAD)
    out = _tc_call(_final_body,
                   jax.ShapeDtypeStruct((N_NODES, D_OUT_PAD), jnp.float32),
                   acc2, y2, dinv)
    return out[:, :W2.shape[1]]
